# Initial kernel scaffold; baseline (speedup 1.0000x reference)
import jax
import jax.numpy as jnp
from jax.experimental import pallas as pl


def kernel(x, edge_index, W1, b1, W2, b2):
    raise NotImplementedError("write your pallas kernel here")


# speedup vs baseline: 22.0855x; 22.0855x over previous
import functools

import jax
import jax.numpy as jnp
from jax import lax
from jax.experimental import pallas as pl
from jax.experimental.pallas import tpu as pltpu
from jax.experimental.pallas import tpu_sc as plsc

N = 10000
E = 320000
D_IN = 128
D_HID = 128
D_OUT = 2
D_PAD = 16

NC = 2
NS = 16
NW = NC * NS
EPT = E // NW
CHUNK = 80
NCHUNK = EPT // CHUNK
ROWS_PT = 1000
N_IO_TILES = N // ROWS_PT

_MESH = plsc.VectorSubcoreMesh(
    core_axis_name="c", subcore_axis_name="s", num_cores=NC, num_subcores=NS
)


def _wid():
    return lax.axis_index("c") * NS + lax.axis_index("s")


@functools.partial(
    pl.kernel,
    out_type=jax.ShapeDtypeStruct((NC, N, D_PAD), jnp.float32),
    mesh=_MESH,
    scratch_types=[
        pltpu.VMEM((NCHUNK, CHUNK), jnp.int32),
        pltpu.VMEM((CHUNK, D_PAD), jnp.float32),
        pltpu.VMEM_SHARED((N, D_PAD), jnp.float32),
    ],
    compiler_params=pltpu.CompilerParams(use_tc_tiling_on_sc=False),
)
def _sc_degree(dst3_hbm, ones_hbm, zeros_hbm, out_hbm, dst_v, ones_v, acc):
    c = lax.axis_index("c")
    s = lax.axis_index("s")
    wid = _wid()

    @pl.when(s < N_IO_TILES)
    def _zero():
        sl = pl.ds(s * ROWS_PT, ROWS_PT)
        pltpu.sync_copy(zeros_hbm.at[sl], acc.at[sl])

    pltpu.sync_copy(dst3_hbm.at[wid], dst_v)
    pltpu.sync_copy(ones_hbm, ones_v)
    plsc.subcore_barrier()

    def body(i, carry):
        pltpu.sync_copy(ones_v, acc.at[dst_v.at[i]], add=True)
        return carry

    lax.fori_loop(0, NCHUNK, body, 0)
    plsc.subcore_barrier()

    @pl.when(s < N_IO_TILES)
    def _out():
        sl = pl.ds(s * ROWS_PT, ROWS_PT)
        pltpu.sync_copy(acc.at[sl], out_hbm.at[c, sl])


def _make_sc_agg(D):
    @functools.partial(
        pl.kernel,
        out_type=jax.ShapeDtypeStruct((NC, N, D), jnp.float32),
        mesh=_MESH,
        scratch_types=[
            pltpu.VMEM((NCHUNK, CHUNK), jnp.int32),
            pltpu.VMEM((NCHUNK, CHUNK), jnp.int32),
            pltpu.VMEM((CHUNK, D), jnp.float32),
            pltpu.VMEM_SHARED((N, D), jnp.float32),
            pltpu.SemaphoreType.DMA,
        ],
        compiler_params=pltpu.CompilerParams(use_tc_tiling_on_sc=False),
    )
    def _sc_agg(hp_hbm, src3_hbm, dst3_hbm, zeros_hbm, out_hbm,
                src_v, dst_v, rows_v, acc, sem):
        c = lax.axis_index("c")
        s = lax.axis_index("s")
        wid = _wid()

        @pl.when(s < N_IO_TILES)
        def _zero():
            sl = pl.ds(s * ROWS_PT, ROWS_PT)
            pltpu.sync_copy(zeros_hbm.at[sl], acc.at[sl])

        pltpu.sync_copy(src3_hbm.at[wid], src_v)
        pltpu.sync_copy(dst3_hbm.at[wid], dst_v)
        plsc.subcore_barrier()

        def body(i, carry):
            pltpu.async_copy(hp_hbm.at[src_v.at[i]], rows_v, sem).wait()
            pltpu.sync_copy(rows_v, acc.at[dst_v.at[i]], add=True)
            return carry

        lax.fori_loop(0, NCHUNK, body, 0)
        plsc.subcore_barrier()

        @pl.when(s < N_IO_TILES)
        def _out():
            sl = pl.ds(s * ROWS_PT, ROWS_PT)
            pltpu.sync_copy(acc.at[sl], out_hbm.at[c, sl])

    return _sc_agg


_sc_agg_128 = _make_sc_agg(D_HID)
_sc_agg_16 = _make_sc_agg(D_PAD)


_BR = 1000
_GRID = N // _BR


def _tc1_body(x_ref, w1_ref, d0_ref, d1_ref, hp_ref, dis_ref):
    deg = d0_ref[...] + d1_ref[...] + 1.0
    dis = lax.rsqrt(deg)
    h = jnp.dot(x_ref[...], w1_ref[...], preferred_element_type=jnp.float32,
                precision=lax.Precision.HIGHEST)
    hp_ref[...] = h * dis
    dis_ref[...] = dis


def _tc1(x, W1, d0, d1):
    return pl.pallas_call(
        _tc1_body,
        grid=(_GRID,),
        in_specs=[
            pl.BlockSpec((_BR, D_IN), lambda i: (i, 0)),
            pl.BlockSpec((D_IN, D_HID), lambda i: (0, 0)),
            pl.BlockSpec((_BR, 1), lambda i: (i, 0)),
            pl.BlockSpec((_BR, 1), lambda i: (i, 0)),
        ],
        out_specs=[
            pl.BlockSpec((_BR, D_HID), lambda i: (i, 0)),
            pl.BlockSpec((_BR, 1), lambda i: (i, 0)),
        ],
        out_shape=[
            jax.ShapeDtypeStruct((N, D_HID), jnp.float32),
            jax.ShapeDtypeStruct((N, 1), jnp.float32),
        ],
    )(x, W1, d0, d1)


def _tc2_body(p0_ref, p1_ref, hp_ref, dis_ref, b1_ref, w2_ref, h2p_ref):
    dis = dis_ref[...]
    z = (p0_ref[...] + p1_ref[...] + hp_ref[...]) * dis + b1_ref[...]
    z = jnp.maximum(z, 0.0)
    h2 = jnp.dot(z, w2_ref[...], preferred_element_type=jnp.float32,
                 precision=lax.Precision.HIGHEST)
    h2p_ref[...] = h2 * dis


def _tc2(p0, p1, hp, dis, b1, W2p):
    return pl.pallas_call(
        _tc2_body,
        grid=(_GRID,),
        in_specs=[
            pl.BlockSpec((_BR, D_HID), lambda i: (i, 0)),
            pl.BlockSpec((_BR, D_HID), lambda i: (i, 0)),
            pl.BlockSpec((_BR, D_HID), lambda i: (i, 0)),
            pl.BlockSpec((_BR, 1), lambda i: (i, 0)),
            pl.BlockSpec((1, D_HID), lambda i: (0, 0)),
            pl.BlockSpec((D_HID, D_PAD), lambda i: (0, 0)),
        ],
        out_specs=pl.BlockSpec((_BR, D_PAD), lambda i: (i, 0)),
        out_shape=jax.ShapeDtypeStruct((N, D_PAD), jnp.float32),
    )(p0, p1, hp, dis, b1, W2p)


def _tc3_body(q0_ref, q1_ref, h2p_ref, dis_ref, b2_ref, out_ref):
    out_ref[...] = (q0_ref[...] + q1_ref[...] + h2p_ref[...]) * dis_ref[...] \
        + b2_ref[...]


def _tc3(q0, q1, h2p, dis, b2p):
    return pl.pallas_call(
        _tc3_body,
        grid=(_GRID,),
        in_specs=[
            pl.BlockSpec((_BR, D_PAD), lambda i: (i, 0)),
            pl.BlockSpec((_BR, D_PAD), lambda i: (i, 0)),
            pl.BlockSpec((_BR, D_PAD), lambda i: (i, 0)),
            pl.BlockSpec((_BR, 1), lambda i: (i, 0)),
            pl.BlockSpec((1, D_PAD), lambda i: (0, 0)),
        ],
        out_specs=pl.BlockSpec((_BR, D_PAD), lambda i: (i, 0)),
        out_shape=jax.ShapeDtypeStruct((N, D_PAD), jnp.float32),
    )(q0, q1, h2p, dis, b2p)


def kernel(x, edge_index, W1, b1, W2, b2):
    src3 = edge_index[0].reshape(NW, NCHUNK, CHUNK)
    dst3 = edge_index[1].reshape(NW, NCHUNK, CHUNK)
    ones16 = jnp.ones((CHUNK, D_PAD), jnp.float32)
    zeros16 = jnp.zeros((N, D_PAD), jnp.float32)
    zeros128 = jnp.zeros((N, D_HID), jnp.float32)
    W2p = jnp.pad(W2, ((0, 0), (0, D_PAD - D_OUT)))
    b1r = b1.reshape(1, D_HID)
    b2p = jnp.pad(b2, (0, D_PAD - D_OUT)).reshape(1, D_PAD)

    degp = _sc_degree(dst3, ones16, zeros16)
    d0 = degp[0, :, 0].reshape(N, 1)
    d1 = degp[1, :, 0].reshape(N, 1)

    hp, dis = _tc1(x, W1, d0, d1)

    aggp = _sc_agg_128(hp, src3, dst3, zeros128)
    h2p = _tc2(aggp[0], aggp[1], hp, dis, b1r, W2p)

    agg2p = _sc_agg_16(h2p, src3, dst3, zeros16)
    out16 = _tc3(agg2p[0], agg2p[1], h2p, dis, b2p)
    return out16[:, :D_OUT]
